# baseline (speedup 1.0000x reference)
import functools

import jax
import jax.numpy as jnp
from jax import lax
from jax.experimental import pallas as pl
from jax.experimental.pallas import tpu as pltpu
from jax.experimental.pallas import tpu_sc as plsc

L = 16
NC, NS = 2, 16
NW = NC * NS
CHUNK = 16


def _floor_to_int(f):
    t = f.astype(jnp.int32)
    return jnp.where(f < 0.0, jnp.int32(-1), t)


def _corner_math(xs, ys, gw, gh):
    fx = xs * float(gw) - 0.5
    fy = ys * float(gh) - 0.5
    ix0 = _floor_to_int(fx)
    iy0 = _floor_to_int(fy)
    wx1 = fx - ix0.astype(jnp.float32)
    wy1 = fy - iy0.astype(jnp.float32)
    wx0 = 1.0 - wx1
    wy0 = 1.0 - wy1
    ix1 = ix0 + 1
    iy1 = iy0 + 1
    vx0 = ix0 >= 0
    vy0 = iy0 >= 0
    vx1 = ix1 <= gw - 1
    vy1 = iy1 <= gh - 1
    cx0 = jnp.maximum(ix0, 0)
    cy0 = jnp.maximum(iy0, 0)
    cx1 = jnp.minimum(ix1, gw - 1)
    cy1 = jnp.minimum(iy1, gh - 1)
    zero = jnp.zeros((L,), jnp.float32)
    idx = [cy0 * gw + cx0, cy0 * gw + cx1, cy1 * gw + cx0, cy1 * gw + cx1]
    wgt = [jnp.where(vx0 & vy0, wx0 * wy0, zero),
           jnp.where(vx1 & vy0, wx1 * wy0, zero),
           jnp.where(vx0 & vy1, wx0 * wy1, zero),
           jnp.where(vx1 & vy1, wx1 * wy1, zero)]
    return idx, wgt


def _make_sc_sampler(B, P, Cin, Hf, Wf_, Hc, Wc, DOUT):
    HWf = Hf * Wf_
    HWc = Hc * Wc
    WPB = NW // B
    PPW = P // WPB
    NCHUNK = PPW // CHUNK
    CG = Cin // L
    mesh = plsc.VectorSubcoreMesh(core_axis_name="c", subcore_axis_name="s")

    @functools.partial(
        pl.kernel,
        mesh=mesh,
        out_type=jax.ShapeDtypeStruct((B * P, DOUT), jnp.float32),
        scratch_types=[
            pltpu.VMEM((2 * PPW,), jnp.float32),
            pltpu.VMEM((HWc,), jnp.float32),
            pltpu.VMEM((PPW,), jnp.int32),
            pltpu.VMEM((PPW,), jnp.int32),
            pltpu.VMEM((PPW,), jnp.int32),
            pltpu.VMEM((PPW,), jnp.int32),
            pltpu.VMEM((PPW,), jnp.float32),
            pltpu.VMEM((PPW,), jnp.float32),
            pltpu.VMEM((PPW,), jnp.float32),
            pltpu.VMEM((PPW,), jnp.float32),
            pltpu.VMEM((PPW,), jnp.float32),
            pltpu.VMEM((CHUNK, Cin), jnp.float32),
            pltpu.VMEM((CHUNK, Cin), jnp.float32),
            pltpu.VMEM((CHUNK, Cin), jnp.float32),
            pltpu.VMEM((CHUNK, Cin), jnp.float32),
            pltpu.VMEM((CHUNK, Cin), jnp.float32),
            pltpu.VMEM((CHUNK, Cin), jnp.float32),
            pltpu.VMEM((CHUNK, Cin), jnp.float32),
            pltpu.VMEM((CHUNK, Cin), jnp.float32),
            pltpu.VMEM((CHUNK, DOUT), jnp.float32),
            pltpu.VMEM((CHUNK, DOUT), jnp.float32),
            pltpu.SemaphoreType.DMA,
            pltpu.SemaphoreType.DMA,
            pltpu.SemaphoreType.DMA,
        ],
        compiler_params=pltpu.CompilerParams(needs_layout_passes=False),
    )
    def sampler(fine_hbm, coarse_hbm, coords_hbm, out_hbm,
                coords_v, ctab_v, i00_v, i01_v, i10_v, i11_v,
                w00_v, w01_v, w10_v, w11_v, cs_v,
                a00_v, a01_v, a10_v, a11_v,
                c00_v, c01_v, c10_v, c11_v,
                acc0_v, acc1_v, gsem0, gsem1, ssem):
        wid = lax.axis_index("s") * NC + lax.axis_index("c")
        b = wid // WPB
        q = wid % WPB

        pltpu.sync_copy(coarse_hbm.at[b], ctab_v)
        pltpu.sync_copy(coords_hbm.at[b, pl.ds(2 * q * PPW, 2 * PPW)],
                        coords_v)

        def idxmath(g, _):
            lanes = jnp.arange(L, dtype=jnp.int32)
            base = 2 * g * L
            sl = pl.ds(g * L, L)
            xs = plsc.load_gather(coords_v, [base + 2 * lanes])
            ys = plsc.load_gather(coords_v, [base + 2 * lanes + 1])
            fidx, fwgt = _corner_math(xs, ys, Wf_, Hf)
            fbase = b * HWf
            i00_v[sl] = fbase + fidx[0]
            i01_v[sl] = fbase + fidx[1]
            i10_v[sl] = fbase + fidx[2]
            i11_v[sl] = fbase + fidx[3]
            w00_v[sl] = fwgt[0]
            w01_v[sl] = fwgt[1]
            w10_v[sl] = fwgt[2]
            w11_v[sl] = fwgt[3]
            cidx, cwgt = _corner_math(xs, ys, Wc, Hc)
            cs_v[sl] = (cwgt[0] * plsc.load_gather(ctab_v, [cidx[0]])
                        + cwgt[1] * plsc.load_gather(ctab_v, [cidx[1]])
                        + cwgt[2] * plsc.load_gather(ctab_v, [cidx[2]])
                        + cwgt[3] * plsc.load_gather(ctab_v, [cidx[3]]))
            return 0

        lax.fori_loop(0, PPW // L, idxmath, 0)

        set0 = (a00_v, a01_v, a10_v, a11_v)
        set1 = (c00_v, c01_v, c10_v, c11_v)

        def fire(ci, bufs, gsem):
            sl = pl.ds(ci * CHUNK, CHUNK)
            pltpu.async_copy(fine_hbm.at[i00_v[sl]], bufs[0], gsem)
            pltpu.async_copy(fine_hbm.at[i01_v[sl]], bufs[1], gsem)
            pltpu.async_copy(fine_hbm.at[i10_v[sl]], bufs[2], gsem)
            pltpu.async_copy(fine_hbm.at[i11_v[sl]], bufs[3], gsem)

        def drain_gathers(bufs, gsem):
            for k in range(4):
                pltpu.make_async_copy(fine_hbm.at[pl.ds(0, CHUNK)],
                                      bufs[k], gsem).wait()

        def combine(ci, bufs, acc_v):
            def one_point(p, _):
                pidx = jnp.full((L,), ci * CHUNK + p, jnp.int32)
                s00 = plsc.load_gather(w00_v, [pidx])
                s01 = plsc.load_gather(w01_v, [pidx])
                s10 = plsc.load_gather(w10_v, [pidx])
                s11 = plsc.load_gather(w11_v, [pidx])
                for g in range(CG):
                    sl = pl.ds(g * L, L)
                    acc_v[p, sl] = (s00 * bufs[0][p, sl] + s01 * bufs[1][p, sl]
                                    + s10 * bufs[2][p, sl]
                                    + s11 * bufs[3][p, sl])
                cs = plsc.load_gather(cs_v, [pidx])
                lane0 = jnp.where(jnp.arange(L, dtype=jnp.int32) == 0,
                                  jnp.full((L,), 1.0, jnp.float32),
                                  jnp.full((L,), 0.0, jnp.float32))
                acc_v[p, pl.ds(Cin, L)] = cs * lane0
                return 0

            lax.fori_loop(0, CHUNK, one_point, 0)

        def store(ci, acc_v):
            row0 = b * P + q * PPW + ci * CHUNK
            pltpu.async_copy(acc_v, out_hbm.at[pl.ds(row0, CHUNK)], ssem)

        def drain_store(acc_v):
            pltpu.make_async_copy(out_hbm.at[pl.ds(0, CHUNK)], acc_v,
                                  ssem).wait()

        fire(0, set0, gsem0)

        def body(j, _):
            ci0 = 2 * j
            fire(ci0 + 1, set1, gsem1)

            @pl.when(j > 0)
            def _():
                drain_store(acc0_v)
                drain_store(acc1_v)

            drain_gathers(set0, gsem0)
            combine(ci0, set0, acc0_v)
            store(ci0, acc0_v)

            @pl.when(ci0 + 2 < NCHUNK)
            def _():
                fire(ci0 + 2, set0, gsem0)

            drain_gathers(set1, gsem1)
            combine(ci0 + 1, set1, acc1_v)
            store(ci0 + 1, acc1_v)
            return 0

        lax.fori_loop(0, NCHUNK // 2, body, 0)
        drain_store(acc0_v)
        drain_store(acc1_v)

    return sampler


def _proj_body(x_ref, w_ref, o_ref):
    o_ref[...] = jax.lax.dot_general(
        x_ref[0], w_ref[...], (((0,), (0,)), ((), ())),
        preferred_element_type=jnp.float32)[None]


def _project_fine(fine, w1f_t, BN=4096):
    B, Cin, HW = fine.shape
    fc = w1f_t.shape[1]
    return pl.pallas_call(
        _proj_body,
        grid=(B, HW // BN),
        in_specs=[
            pl.BlockSpec((1, Cin, BN), lambda b, j: (b, 0, j)),
            pl.BlockSpec((Cin, fc), lambda b, j: (0, 0)),
        ],
        out_specs=pl.BlockSpec((1, BN, fc), lambda b, j: (b, j, 0)),
        out_shape=jax.ShapeDtypeStruct((B, HW, fc), jnp.float32),
        compiler_params=pltpu.CompilerParams(
            fuse_transposed_lhs_in_matmul=True),
    )(fine, w1f_t)


def _mlp_body(x_ref, w1c_ref, b1_ref, w2_ref, b2_ref, w3_ref, b3_ref,
              wf_ref, bf_ref, o_ref, *, fc):
    x = x_ref[...]
    h = x[:, :fc] + x[:, fc:fc + 1] * w1c_ref[...] + b1_ref[...]
    h = jnp.maximum(h, 0.0)
    h = jnp.dot(h, w2_ref[...], preferred_element_type=jnp.float32)
    h = jnp.maximum(h + b2_ref[...], 0.0)
    h = jnp.dot(h, w3_ref[...], preferred_element_type=jnp.float32)
    h = jnp.maximum(h + b3_ref[...], 0.0)
    o_ref[...] = (jnp.dot(h, wf_ref[...], preferred_element_type=jnp.float32)
                  + bf_ref[...])


def _mlp(x, w1c, b1, w2t, b2, w3t, b3, wft, bf, BM=2048):
    N, K = x.shape
    fc = w2t.shape[0]
    grid = (N // BM,)
    full = lambda i: (0, 0)
    return pl.pallas_call(
        functools.partial(_mlp_body, fc=fc),
        grid=grid,
        in_specs=[
            pl.BlockSpec((BM, K), lambda i: (i, 0)),
            pl.BlockSpec((1, fc), full),
            pl.BlockSpec((1, fc), full),
            pl.BlockSpec((fc, fc), full),
            pl.BlockSpec((1, fc), full),
            pl.BlockSpec((fc, fc), full),
            pl.BlockSpec((1, fc), full),
            pl.BlockSpec((fc, 1), full),
            pl.BlockSpec((1, 1), full),
        ],
        out_specs=pl.BlockSpec((BM, 1), lambda i: (i, 0)),
        out_shape=jax.ShapeDtypeStruct((N, 1), jnp.float32),
    )(x, w1c, b1, w2t, b2, w3t, b3, wft, bf)


def kernel(coarse_logits, fine_features, point_coords,
           W1, b1, W2, b2, W3, b3, Wf, bf):
    B, Cout, Hc, Wc = coarse_logits.shape
    _, Cin, Hf, Wf_ = fine_features.shape
    P = point_coords.shape[1]
    fc = W1.shape[0]
    DOUT = fc + L

    fine_r = fine_features.reshape(B, Cin, Hf * Wf_)
    coarse_flat = coarse_logits.reshape(B, Hc * Wc)
    coords_flat = point_coords.reshape(B, 2 * P)

    w1f_t = W1[:, :Cin].T
    w1c = W1[:, Cin].reshape(1, fc)

    NGRP = 2
    GB = B // NGRP
    sampler = _make_sc_sampler(GB, P, fc, Hf, Wf_, Hc, Wc, DOUT)
    ys = []
    for g in range(NGRP):
        bs = slice(g * GB, (g + 1) * GB)
        a_tab = _project_fine(fine_r[bs], w1f_t).reshape(GB * Hf * Wf_, fc)
        sampled = sampler(a_tab, coarse_flat[bs], coords_flat[bs])
        ys.append(_mlp(sampled, w1c, b1.reshape(1, fc), W2.T,
                       b2.reshape(1, fc), W3.T, b3.reshape(1, fc),
                       Wf.T, bf.reshape(1, 1)))
    y = jnp.concatenate(ys, axis=0)
    return y.reshape(B, P, Cout).transpose(0, 2, 1)


# speedup vs baseline: 1.1715x; 1.1715x over previous
import functools

import jax
import jax.numpy as jnp
from jax import lax
from jax.experimental import pallas as pl
from jax.experimental.pallas import tpu as pltpu
from jax.experimental.pallas import tpu_sc as plsc

L = 16
NC, NS = 2, 16
NW = NC * NS
CHUNK = 16


def _floor_to_int(f):
    t = f.astype(jnp.int32)
    return jnp.where(f < 0.0, jnp.int32(-1), t)


def _corner_math(xs, ys, gw, gh):
    fx = xs * float(gw) - 0.5
    fy = ys * float(gh) - 0.5
    ix0 = _floor_to_int(fx)
    iy0 = _floor_to_int(fy)
    wx1 = fx - ix0.astype(jnp.float32)
    wy1 = fy - iy0.astype(jnp.float32)
    wx0 = 1.0 - wx1
    wy0 = 1.0 - wy1
    ix1 = ix0 + 1
    iy1 = iy0 + 1
    vx0 = ix0 >= 0
    vy0 = iy0 >= 0
    vx1 = ix1 <= gw - 1
    vy1 = iy1 <= gh - 1
    cx0 = jnp.maximum(ix0, 0)
    cy0 = jnp.maximum(iy0, 0)
    cx1 = jnp.minimum(ix1, gw - 1)
    cy1 = jnp.minimum(iy1, gh - 1)
    zero = jnp.zeros((L,), jnp.float32)
    idx = [cy0 * gw + cx0, cy0 * gw + cx1, cy1 * gw + cx0, cy1 * gw + cx1]
    wgt = [jnp.where(vx0 & vy0, wx0 * wy0, zero),
           jnp.where(vx1 & vy0, wx1 * wy0, zero),
           jnp.where(vx0 & vy1, wx0 * wy1, zero),
           jnp.where(vx1 & vy1, wx1 * wy1, zero)]
    return idx, wgt


def _make_sc_sampler(B, P, Cin, Hf, Wf_, Hc, Wc, DOUT):
    HWf = Hf * Wf_
    HWc = Hc * Wc
    WPB = NW // B
    PPW = P // WPB
    NCHUNK = PPW // CHUNK
    CG = Cin // L
    mesh = plsc.VectorSubcoreMesh(core_axis_name="c", subcore_axis_name="s")

    @functools.partial(
        pl.kernel,
        mesh=mesh,
        out_type=jax.ShapeDtypeStruct((B * P, DOUT), jnp.float32),
        scratch_types=[
            pltpu.VMEM((2 * PPW,), jnp.float32),
            pltpu.VMEM((HWc,), jnp.float32),
            pltpu.VMEM((PPW,), jnp.int32),
            pltpu.VMEM((PPW,), jnp.int32),
            pltpu.VMEM((PPW,), jnp.int32),
            pltpu.VMEM((PPW,), jnp.int32),
            pltpu.VMEM((PPW,), jnp.float32),
            pltpu.VMEM((PPW,), jnp.float32),
            pltpu.VMEM((PPW,), jnp.float32),
            pltpu.VMEM((PPW,), jnp.float32),
            pltpu.VMEM((PPW,), jnp.float32),
            pltpu.VMEM((CHUNK, Cin), jnp.float32),
            pltpu.VMEM((CHUNK, Cin), jnp.float32),
            pltpu.VMEM((CHUNK, Cin), jnp.float32),
            pltpu.VMEM((CHUNK, Cin), jnp.float32),
            pltpu.VMEM((CHUNK, Cin), jnp.float32),
            pltpu.VMEM((CHUNK, Cin), jnp.float32),
            pltpu.VMEM((CHUNK, Cin), jnp.float32),
            pltpu.VMEM((CHUNK, Cin), jnp.float32),
            pltpu.VMEM((CHUNK, DOUT), jnp.float32),
            pltpu.VMEM((CHUNK, DOUT), jnp.float32),
            pltpu.SemaphoreType.DMA,
            pltpu.SemaphoreType.DMA,
            pltpu.SemaphoreType.DMA,
        ],
        compiler_params=pltpu.CompilerParams(needs_layout_passes=False),
    )
    def sampler(fine_hbm, coarse_hbm, coords_hbm, out_hbm,
                coords_v, ctab_v, i00_v, i01_v, i10_v, i11_v,
                w00_v, w01_v, w10_v, w11_v, cs_v,
                a00_v, a01_v, a10_v, a11_v,
                c00_v, c01_v, c10_v, c11_v,
                acc0_v, acc1_v, gsem0, gsem1, ssem):
        wid = lax.axis_index("s") * NC + lax.axis_index("c")
        b = wid // WPB
        q = wid % WPB

        pltpu.sync_copy(coarse_hbm.at[b], ctab_v)
        pltpu.sync_copy(coords_hbm.at[b, pl.ds(2 * q * PPW, 2 * PPW)],
                        coords_v)

        def idxmath(g, _):
            lanes = jnp.arange(L, dtype=jnp.int32)
            base = 2 * g * L
            sl = pl.ds(g * L, L)
            xs = plsc.load_gather(coords_v, [base + 2 * lanes])
            ys = plsc.load_gather(coords_v, [base + 2 * lanes + 1])
            fidx, fwgt = _corner_math(xs, ys, Wf_, Hf)
            fbase = b * HWf
            i00_v[sl] = fbase + fidx[0]
            i01_v[sl] = fbase + fidx[1]
            i10_v[sl] = fbase + fidx[2]
            i11_v[sl] = fbase + fidx[3]
            w00_v[sl] = fwgt[0]
            w01_v[sl] = fwgt[1]
            w10_v[sl] = fwgt[2]
            w11_v[sl] = fwgt[3]
            cidx, cwgt = _corner_math(xs, ys, Wc, Hc)
            cs_v[sl] = (cwgt[0] * plsc.load_gather(ctab_v, [cidx[0]])
                        + cwgt[1] * plsc.load_gather(ctab_v, [cidx[1]])
                        + cwgt[2] * plsc.load_gather(ctab_v, [cidx[2]])
                        + cwgt[3] * plsc.load_gather(ctab_v, [cidx[3]]))
            return 0

        lax.fori_loop(0, PPW // L, idxmath, 0)

        set0 = (a00_v, a01_v, a10_v, a11_v)
        set1 = (c00_v, c01_v, c10_v, c11_v)

        def fire(ci, bufs, gsem):
            sl = pl.ds(ci * CHUNK, CHUNK)
            pltpu.async_copy(fine_hbm.at[i00_v[sl]], bufs[0], gsem)
            pltpu.async_copy(fine_hbm.at[i01_v[sl]], bufs[1], gsem)
            pltpu.async_copy(fine_hbm.at[i10_v[sl]], bufs[2], gsem)
            pltpu.async_copy(fine_hbm.at[i11_v[sl]], bufs[3], gsem)

        def drain_gathers(bufs, gsem):
            for k in range(4):
                pltpu.make_async_copy(fine_hbm.at[pl.ds(0, CHUNK)],
                                      bufs[k], gsem).wait()

        def combine(ci, bufs, acc_v):
            def one_point(p, _):
                pidx = jnp.full((L,), ci * CHUNK + p, jnp.int32)
                s00 = plsc.load_gather(w00_v, [pidx])
                s01 = plsc.load_gather(w01_v, [pidx])
                s10 = plsc.load_gather(w10_v, [pidx])
                s11 = plsc.load_gather(w11_v, [pidx])
                for g in range(CG):
                    sl = pl.ds(g * L, L)
                    acc_v[p, sl] = (s00 * bufs[0][p, sl] + s01 * bufs[1][p, sl]
                                    + s10 * bufs[2][p, sl]
                                    + s11 * bufs[3][p, sl])
                cs = plsc.load_gather(cs_v, [pidx])
                lane0 = jnp.where(jnp.arange(L, dtype=jnp.int32) == 0,
                                  jnp.full((L,), 1.0, jnp.float32),
                                  jnp.full((L,), 0.0, jnp.float32))
                acc_v[p, pl.ds(Cin, L)] = cs * lane0
                return 0

            lax.fori_loop(0, CHUNK, one_point, 0)

        def store(ci, acc_v):
            row0 = b * P + q * PPW + ci * CHUNK
            pltpu.async_copy(acc_v, out_hbm.at[pl.ds(row0, CHUNK)], ssem)

        def drain_store(acc_v):
            pltpu.make_async_copy(out_hbm.at[pl.ds(0, CHUNK)], acc_v,
                                  ssem).wait()

        fire(0, set0, gsem0)

        def body(j, _):
            ci0 = 2 * j
            fire(ci0 + 1, set1, gsem1)

            @pl.when(j > 0)
            def _():
                drain_store(acc0_v)
                drain_store(acc1_v)

            drain_gathers(set0, gsem0)
            combine(ci0, set0, acc0_v)
            store(ci0, acc0_v)

            @pl.when(ci0 + 2 < NCHUNK)
            def _():
                fire(ci0 + 2, set0, gsem0)

            drain_gathers(set1, gsem1)
            combine(ci0 + 1, set1, acc1_v)
            store(ci0 + 1, acc1_v)
            return 0

        lax.fori_loop(0, NCHUNK // 2, body, 0)
        drain_store(acc0_v)
        drain_store(acc1_v)

    return sampler


def _mlp_body(x_ref, w1_ref, b1_ref, w2_ref, b2_ref, w3_ref, b3_ref,
              wf_ref, bf_ref, o_ref):
    h = jnp.dot(x_ref[...], w1_ref[...], preferred_element_type=jnp.float32)
    h = jnp.maximum(h + b1_ref[...], 0.0)
    h = jnp.dot(h, w2_ref[...], preferred_element_type=jnp.float32)
    h = jnp.maximum(h + b2_ref[...], 0.0)
    h = jnp.dot(h, w3_ref[...], preferred_element_type=jnp.float32)
    h = jnp.maximum(h + b3_ref[...], 0.0)
    o_ref[...] = (jnp.dot(h, wf_ref[...], preferred_element_type=jnp.float32)
                  + bf_ref[...])


def _mlp(x, w1t, b1, w2t, b2, w3t, b3, wft, bf, BM=2048):
    N, K = x.shape
    fc = w2t.shape[0]
    grid = (N // BM,)
    full = lambda i: (0, 0)
    return pl.pallas_call(
        _mlp_body,
        grid=grid,
        in_specs=[
            pl.BlockSpec((BM, K), lambda i: (i, 0)),
            pl.BlockSpec((K, fc), full),
            pl.BlockSpec((1, fc), full),
            pl.BlockSpec((fc, fc), full),
            pl.BlockSpec((1, fc), full),
            pl.BlockSpec((fc, fc), full),
            pl.BlockSpec((1, fc), full),
            pl.BlockSpec((fc, 1), full),
            pl.BlockSpec((1, 1), full),
        ],
        out_specs=pl.BlockSpec((BM, 1), lambda i: (i, 0)),
        out_shape=jax.ShapeDtypeStruct((N, 1), jnp.float32),
    )(x, w1t, b1, w2t, b2, w3t, b3, wft, bf)


def kernel(coarse_logits, fine_features, point_coords,
           W1, b1, W2, b2, W3, b3, Wf, bf):
    B, Cout, Hc, Wc = coarse_logits.shape
    _, Cin, Hf, Wf_ = fine_features.shape
    P = point_coords.shape[1]
    fc = W1.shape[0]
    DOUT = Cin + L

    fine_t = fine_features.reshape(B, Cin, Hf * Wf_)
    fine_t = fine_t.transpose(0, 2, 1).reshape(B * Hf * Wf_, Cin)
    coarse_flat = coarse_logits.reshape(B, Hc * Wc)
    coords_flat = point_coords.reshape(B, 2 * P)

    sampler = _make_sc_sampler(B, P, Cin, Hf, Wf_, Hc, Wc, DOUT)
    sampled = sampler(fine_t, coarse_flat, coords_flat)

    w1t = jnp.concatenate(
        [W1.T, jnp.zeros((DOUT - W1.shape[1], fc), jnp.float32)], axis=0)
    y = _mlp(sampled, w1t, b1.reshape(1, fc), W2.T, b2.reshape(1, fc),
             W3.T, b3.reshape(1, fc), Wf.T, bf.reshape(1, 1))
    return y.reshape(B, P, Cout).transpose(0, 2, 1)
